# Initial kernel scaffold; baseline (speedup 1.0000x reference)
import jax
import jax.numpy as jnp
from jax.experimental import pallas as pl


def kernel(atomic_numbers, positions, idx_i, idx_j, emb_table, rbf_centers, rbf_widths):
    raise NotImplementedError("write your pallas kernel here")


# speedup vs baseline: 4.8178x; 4.8178x over previous
import functools

import jax
import jax.numpy as jnp
from jax import lax
from jax.experimental import pallas as pl
from jax.experimental.pallas import tpu as pltpu
from jax.experimental.pallas import tpu_sc as plsc

N_ATOMS = 10000
N_PAIRS = 640000
N_ATOMBASIS = 128
N_RBF = 64
CUTOFF = 8.0

NC, NS = 2, 16
NW = NC * NS
PAIRS_PER_W = N_PAIRS // NW
PAIR_ROUNDS = 2
PAIRS_PER_ROUND = PAIRS_PER_W // PAIR_ROUNDS
FEAT_PAD = 10240
FEAT_PER_W = FEAT_PAD // NW
FEAT_CHUNK = 80


def _sc_sparse(pos_flat, idx_i, idx_j, an_pad, emb_table):
    mesh = plsc.VectorSubcoreMesh(core_axis_name="c", subcore_axis_name="s")

    @functools.partial(
        pl.kernel,
        mesh=mesh,
        compiler_params=pltpu.CompilerParams(needs_layout_passes=False),
        out_type=(
            jax.ShapeDtypeStruct((N_PAIRS,), jnp.float32),
            jax.ShapeDtypeStruct((FEAT_PAD, N_ATOMBASIS), jnp.float32),
        ),
        scratch_types=[
            pltpu.VMEM((3 * N_ATOMS,), jnp.float32),
            pltpu.VMEM((PAIRS_PER_ROUND,), jnp.int32),
            pltpu.VMEM((PAIRS_PER_ROUND,), jnp.int32),
            pltpu.VMEM((PAIRS_PER_ROUND,), jnp.float32),
            pltpu.VMEM((FEAT_PER_W,), jnp.int32),
            pltpu.VMEM((FEAT_PER_W, N_ATOMBASIS), jnp.float32),
            pltpu.SemaphoreType.DMA,
        ],
    )
    def k(pos_hbm, ii_hbm, jj_hbm, an_hbm, emb_hbm, d2_hbm, feat_hbm,
          pos_v, ii_v, jj_v, d2_v, an_v, rows_v, sem):
        wid = lax.axis_index("s") * NC + lax.axis_index("c")

        fbase = wid * FEAT_PER_W
        pltpu.sync_copy(an_hbm.at[pl.ds(fbase, FEAT_PER_W)], an_v)
        for f in range(FEAT_PER_W // FEAT_CHUNK):
            pltpu.async_copy(
                emb_hbm.at[an_v.at[pl.ds(f * FEAT_CHUNK, FEAT_CHUNK)]],
                rows_v.at[pl.ds(f * FEAT_CHUNK, FEAT_CHUNK)],
                sem,
            ).wait()
        pltpu.sync_copy(rows_v, feat_hbm.at[pl.ds(fbase, FEAT_PER_W)])

        pltpu.sync_copy(pos_hbm, pos_v)
        for r in range(PAIR_ROUNDS):
            base = wid * PAIRS_PER_W + r * PAIRS_PER_ROUND
            pltpu.sync_copy(ii_hbm.at[pl.ds(base, PAIRS_PER_ROUND)], ii_v)
            pltpu.sync_copy(jj_hbm.at[pl.ds(base, PAIRS_PER_ROUND)], jj_v)

            def body(t, carry):
                off = t * 16
                ii3 = ii_v[pl.ds(off, 16)] * 3
                jj3 = jj_v[pl.ds(off, 16)] * 3
                dx = (plsc.load_gather(pos_v, [jj3])
                      - plsc.load_gather(pos_v, [ii3]))
                dy = (plsc.load_gather(pos_v, [jj3 + 1])
                      - plsc.load_gather(pos_v, [ii3 + 1]))
                dz = (plsc.load_gather(pos_v, [jj3 + 2])
                      - plsc.load_gather(pos_v, [ii3 + 2]))
                d2_v[pl.ds(off, 16)] = dx * dx + dy * dy + dz * dz
                return carry

            lax.fori_loop(0, PAIRS_PER_ROUND // 16, body, 0)
            pltpu.sync_copy(d2_v, d2_hbm.at[pl.ds(base, PAIRS_PER_ROUND)])

    return k(pos_flat, idx_i, idx_j, an_pad, emb_table)


def _tc_dist(d2r):
    def body(d2_ref, dist_ref, cut_ref, xp_ref):
        d2b = d2_ref[...]
        d = jnp.sqrt(d2b)
        dist_ref[...] = d
        x = d * (1.0 / CUTOFF)
        x3 = x * x * x
        fc = 1.0 - 10.0 * x3 + 15.0 * x3 * x - 6.0 * x3 * x * x
        cut_ref[...] = jnp.where(d < CUTOFF, fc, 0.0)
        xp_ref[...] = jnp.exp(-d)

    R = 40
    n = d2r.shape[0] // R
    out = jax.ShapeDtypeStruct(d2r.shape, jnp.float32)
    return pl.pallas_call(
        body,
        grid=(n,),
        in_specs=[pl.BlockSpec((R, 128), lambda i: (i, 0))],
        out_specs=[pl.BlockSpec((R, 128), lambda i: (i, 0))] * 3,
        out_shape=(out, out, out),
    )(d2r)


def _tc_rbf(xp2, c128, wn128):
    R = 512

    def body(x_ref, c_ref, w_ref, out_ref):
        x = x_ref[...]
        xe = jnp.broadcast_to(x[:, 0:1], (R, N_RBF))
        xo = jnp.broadcast_to(x[:, 1:2], (R, N_RBF))
        xcat = jnp.concatenate([xe, xo], axis=1)
        t = xcat - c_ref[...]
        out_ref[...] = jnp.exp(w_ref[...] * t * t)

    n = xp2.shape[0] // R
    return pl.pallas_call(
        body,
        grid=(n,),
        in_specs=[
            pl.BlockSpec((R, 2), lambda i: (i, 0)),
            pl.BlockSpec((1, 128), lambda i: (0, 0)),
            pl.BlockSpec((1, 128), lambda i: (0, 0)),
        ],
        out_specs=pl.BlockSpec((R, 128), lambda i: (i, 0)),
        out_shape=jax.ShapeDtypeStruct((xp2.shape[0], 128), jnp.float32),
    )(xp2, c128, wn128)


def kernel(atomic_numbers, positions, idx_i, idx_j, emb_table, rbf_centers,
           rbf_widths):
    pos_flat = positions.reshape(-1)
    an_pad = jnp.concatenate(
        [atomic_numbers.astype(jnp.int32),
         jnp.zeros((FEAT_PAD - N_ATOMS,), jnp.int32)])

    d2, feat_pad = _sc_sparse(pos_flat, idx_i.astype(jnp.int32),
                              idx_j.astype(jnp.int32), an_pad, emb_table)

    dist_r, cut_r, xp_r = _tc_dist(d2.reshape(N_PAIRS // 128, 128))

    c128 = jnp.concatenate([rbf_centers, rbf_centers]).reshape(1, 128)
    wn128 = (-jnp.concatenate([rbf_widths, rbf_widths])).reshape(1, 128)
    rbf2 = _tc_rbf(xp_r.reshape(N_PAIRS // 2, 2), c128, wn128)

    features = feat_pad[:N_ATOMS]
    distances = dist_r.reshape(N_PAIRS)
    cutoffs = cut_r.reshape(N_PAIRS)
    rbfs = rbf2.reshape(N_PAIRS, N_RBF)
    return (features, distances, cutoffs, rbfs)


# speedup vs baseline: 7.1550x; 1.4851x over previous
import functools

import jax
import jax.numpy as jnp
from jax import lax
from jax.experimental import pallas as pl
from jax.experimental.pallas import tpu as pltpu
from jax.experimental.pallas import tpu_sc as plsc

N_ATOMS = 10000
N_PAIRS = 640000
N_ATOMBASIS = 128
N_RBF = 64
CUTOFF = 8.0

NC, NS = 2, 16
NW = NC * NS
PAIRS_PER_W = N_PAIRS // NW
PAIR_ROUNDS = 2
PAIRS_PER_ROUND = PAIRS_PER_W // PAIR_ROUNDS
FEAT_PAD = 10240
FEAT_PER_W = FEAT_PAD // NW
FEAT_CHUNK = 80


def _sc_sparse(pos_flat, idx_i, idx_j, an_pad, emb_table):
    mesh = plsc.VectorSubcoreMesh(core_axis_name="c", subcore_axis_name="s")

    @functools.partial(
        pl.kernel,
        mesh=mesh,
        compiler_params=pltpu.CompilerParams(needs_layout_passes=False),
        out_type=(
            jax.ShapeDtypeStruct((N_PAIRS,), jnp.float32),
            jax.ShapeDtypeStruct((FEAT_PAD, N_ATOMBASIS), jnp.float32),
        ),
        scratch_types=[
            pltpu.VMEM((3 * N_ATOMS,), jnp.float32),
            pltpu.VMEM((PAIRS_PER_ROUND,), jnp.int32),
            pltpu.VMEM((PAIRS_PER_ROUND,), jnp.int32),
            pltpu.VMEM((PAIRS_PER_ROUND,), jnp.float32),
            pltpu.VMEM((FEAT_PER_W,), jnp.int32),
            pltpu.VMEM((FEAT_PER_W, N_ATOMBASIS), jnp.float32),
            pltpu.SemaphoreType.DMA,
        ],
    )
    def k(pos_hbm, ii_hbm, jj_hbm, an_hbm, emb_hbm, d2_hbm, feat_hbm,
          pos_v, ii_v, jj_v, d2_v, an_v, rows_v, sem):
        wid = lax.axis_index("s") * NC + lax.axis_index("c")

        fbase = wid * FEAT_PER_W
        pltpu.sync_copy(an_hbm.at[pl.ds(fbase, FEAT_PER_W)], an_v)
        for f in range(FEAT_PER_W // FEAT_CHUNK):
            pltpu.async_copy(
                emb_hbm.at[an_v.at[pl.ds(f * FEAT_CHUNK, FEAT_CHUNK)]],
                rows_v.at[pl.ds(f * FEAT_CHUNK, FEAT_CHUNK)],
                sem,
            ).wait()
        pltpu.sync_copy(rows_v, feat_hbm.at[pl.ds(fbase, FEAT_PER_W)])

        pltpu.sync_copy(pos_hbm, pos_v)
        for r in range(PAIR_ROUNDS):
            base = wid * PAIRS_PER_W + r * PAIRS_PER_ROUND
            pltpu.sync_copy(ii_hbm.at[pl.ds(base, PAIRS_PER_ROUND)], ii_v)
            pltpu.sync_copy(jj_hbm.at[pl.ds(base, PAIRS_PER_ROUND)], jj_v)

            @plsc.parallel_loop(0, PAIRS_PER_ROUND, step=16, unroll=4)
            def body(off):
                ii3 = ii_v[pl.ds(off, 16)] * 3
                jj3 = jj_v[pl.ds(off, 16)] * 3
                dx = (plsc.load_gather(pos_v, [jj3])
                      - plsc.load_gather(pos_v, [ii3]))
                dy = (plsc.load_gather(pos_v, [jj3 + 1])
                      - plsc.load_gather(pos_v, [ii3 + 1]))
                dz = (plsc.load_gather(pos_v, [jj3 + 2])
                      - plsc.load_gather(pos_v, [ii3 + 2]))
                d2_v[pl.ds(off, 16)] = dx * dx + dy * dy + dz * dz
            pltpu.sync_copy(d2_v, d2_hbm.at[pl.ds(base, PAIRS_PER_ROUND)])

    return k(pos_flat, idx_i, idx_j, an_pad, emb_table)


def _tc_dist(d2r):
    def body(d2_ref, dist_ref, cut_ref, xp_ref):
        d2b = d2_ref[...]
        d = jnp.sqrt(d2b)
        dist_ref[...] = d
        x = d * (1.0 / CUTOFF)
        x3 = x * x * x
        fc = 1.0 - 10.0 * x3 + 15.0 * x3 * x - 6.0 * x3 * x * x
        cut_ref[...] = jnp.where(d < CUTOFF, fc, 0.0)
        xp_ref[...] = jnp.exp(-d)

    R = 40
    n = d2r.shape[0] // R
    out = jax.ShapeDtypeStruct(d2r.shape, jnp.float32)
    return pl.pallas_call(
        body,
        grid=(n,),
        in_specs=[pl.BlockSpec((R, 128), lambda i: (i, 0))],
        out_specs=[pl.BlockSpec((R, 128), lambda i: (i, 0))] * 3,
        out_shape=(out, out, out),
    )(d2r)


def _tc_rbf(xp_r, c64, wn64):
    P = 1024

    def body(x_ref, c_ref, w_ref, out_ref):
        x8 = x_ref[...]
        xt = jnp.transpose(x8)
        c = c_ref[...]
        w = w_ref[...]
        for a in range(8):
            xb = jnp.broadcast_to(xt[:, a:a + 1], (128, N_RBF))
            t = xb - c
            out_ref[pl.ds(a * 128, 128), :] = jnp.exp(w * t * t)

    n = N_PAIRS // P
    return pl.pallas_call(
        body,
        grid=(n,),
        in_specs=[
            pl.BlockSpec((P // 128, 128), lambda i: (i, 0)),
            pl.BlockSpec((1, N_RBF), lambda i: (0, 0)),
            pl.BlockSpec((1, N_RBF), lambda i: (0, 0)),
        ],
        out_specs=pl.BlockSpec((P, N_RBF), lambda i: (i, 0)),
        out_shape=jax.ShapeDtypeStruct((N_PAIRS, N_RBF), jnp.float32),
    )(xp_r, c64, wn64)


def kernel(atomic_numbers, positions, idx_i, idx_j, emb_table, rbf_centers,
           rbf_widths):
    pos_flat = positions.reshape(-1)
    an_pad = jnp.concatenate(
        [atomic_numbers.astype(jnp.int32),
         jnp.zeros((FEAT_PAD - N_ATOMS,), jnp.int32)])

    d2, feat_pad = _sc_sparse(pos_flat, idx_i.astype(jnp.int32),
                              idx_j.astype(jnp.int32), an_pad, emb_table)

    dist_r, cut_r, xp_r = _tc_dist(d2.reshape(N_PAIRS // 128, 128))

    c64 = rbf_centers.reshape(1, N_RBF)
    wn64 = (-rbf_widths).reshape(1, N_RBF)
    rbfs = _tc_rbf(xp_r, c64, wn64)

    features = feat_pad[:N_ATOMS]
    distances = dist_r.reshape(N_PAIRS)
    cutoffs = cut_r.reshape(N_PAIRS)
    return (features, distances, cutoffs, rbfs)
